# Initial kernel scaffold; baseline (speedup 1.0000x reference)
import jax
import jax.numpy as jnp
from jax.experimental import pallas as pl


def kernel(x, n_node_edge_index, Wl, Wr, att, b_gat, W1, b1, W2, b2, g1, be1, g2, be2):
    raise NotImplementedError("write your pallas kernel here")


# speedup vs baseline: 1.0114x; 1.0114x over previous
import functools

import jax
import jax.numpy as jnp
from jax.experimental import pallas as pl
from jax.experimental.pallas import tpu as pltpu

B, T, N, D = 4, 12, 1000, 64
H, FH = 8, 8
FF = 256
E = 16000
NEG_SLOPE = 0.2
BT = B * T
NTOT = BT * N

_ROWS = 48000
_BLK = 3000


def _mm_kernel(x_ref, wl_ref, wr_ref, xl_ref, xr_ref):
    x = x_ref[...]
    xl_ref[...] = jax.lax.dot_general(
        x, wl_ref[...], (((1,), (0,)), ((), ())),
        preferred_element_type=jnp.float32,
        precision=jax.lax.Precision.HIGHEST)
    xr_ref[...] = jax.lax.dot_general(
        x, wr_ref[...], (((1,), (0,)), ((), ())),
        preferred_element_type=jnp.float32,
        precision=jax.lax.Precision.HIGHEST)


def _proj(xf, Wl, Wr):
    grid = _ROWS // _BLK
    return pl.pallas_call(
        _mm_kernel,
        grid=(grid,),
        in_specs=[
            pl.BlockSpec((_BLK, D), lambda i: (i, 0)),
            pl.BlockSpec((D, D), lambda i: (0, 0)),
            pl.BlockSpec((D, D), lambda i: (0, 0)),
        ],
        out_specs=[
            pl.BlockSpec((_BLK, D), lambda i: (i, 0)),
            pl.BlockSpec((_BLK, D), lambda i: (i, 0)),
        ],
        out_shape=[
            jax.ShapeDtypeStruct((_ROWS, D), jnp.float32),
            jax.ShapeDtypeStruct((_ROWS, D), jnp.float32),
        ],
    )(xf, Wl, Wr)


def _ln(v, g, b):
    mu = jnp.mean(v, axis=-1, keepdims=True)
    var = jnp.mean((v - mu) ** 2, axis=-1, keepdims=True)
    return (v - mu) * jax.lax.rsqrt(var + 1e-5) * g + b


def _ffn_kernel(x_ref, agg_ref, w1_ref, b1_ref, w2_ref, b2_ref,
                g1_ref, be1_ref, g2_ref, be2_ref, bg_ref, o_ref):
    x = x_ref[...]
    out = _ln(x + agg_ref[...] + bg_ref[...], g1_ref[...], be1_ref[...])
    h = jax.lax.dot_general(out, w1_ref[...], (((1,), (0,)), ((), ())),
                            preferred_element_type=jnp.float32,
                            precision=jax.lax.Precision.HIGHEST)
    h = jnp.maximum(h + b1_ref[...], 0.0)
    ff = jax.lax.dot_general(h, w2_ref[...], (((1,), (0,)), ((), ())),
                             preferred_element_type=jnp.float32,
                             precision=jax.lax.Precision.HIGHEST)
    ff = ff + b2_ref[...]
    o_ref[...] = _ln(out + ff, g2_ref[...], be2_ref[...])


def _ffn(xf, agg, b_gat, W1, b1, W2, b2, g1, be1, g2, be2):
    grid = _ROWS // _BLK
    vec = lambda: pl.BlockSpec((1, None), lambda i: (0, 0))
    return pl.pallas_call(
        _ffn_kernel,
        grid=(grid,),
        in_specs=[
            pl.BlockSpec((_BLK, D), lambda i: (i, 0)),
            pl.BlockSpec((_BLK, D), lambda i: (i, 0)),
            pl.BlockSpec((D, FF), lambda i: (0, 0)),
            pl.BlockSpec((1, FF), lambda i: (0, 0)),
            pl.BlockSpec((FF, D), lambda i: (0, 0)),
            pl.BlockSpec((1, D), lambda i: (0, 0)),
            pl.BlockSpec((1, D), lambda i: (0, 0)),
            pl.BlockSpec((1, D), lambda i: (0, 0)),
            pl.BlockSpec((1, D), lambda i: (0, 0)),
            pl.BlockSpec((1, D), lambda i: (0, 0)),
            pl.BlockSpec((1, D), lambda i: (0, 0)),
        ],
        out_specs=pl.BlockSpec((_BLK, D), lambda i: (i, 0)),
        out_shape=jax.ShapeDtypeStruct((_ROWS, D), jnp.float32),
    )(xf, agg, W1, b1.reshape(1, FF), W2, b2.reshape(1, D),
      g1.reshape(1, D), be1.reshape(1, D), g2.reshape(1, D),
      be2.reshape(1, D), b_gat.reshape(1, D))


def _edge_phase(xl, xr, att, src, dst):
    offsets = jnp.arange(BT, dtype=jnp.int32) * N
    src_all = (src[None, :] + offsets[:, None]).reshape(-1)
    dst_all = (dst[None, :] + offsets[:, None]).reshape(-1)
    xl = xl.reshape(NTOT, H, FH)
    xr = xr.reshape(NTOT, H, FH)
    e = xl[src_all] + xr[dst_all]
    e = jnp.where(e > 0, e, NEG_SLOPE * e)
    logits = jnp.sum(e * att[None, :, :], axis=-1)
    m = jax.ops.segment_max(logits, dst_all, num_segments=NTOT)
    p = jnp.exp(logits - m[dst_all])
    denom = jax.ops.segment_sum(p, dst_all, num_segments=NTOT)
    alpha = p / (denom[dst_all] + 1e-16)
    msg = xl[src_all] * alpha[:, :, None]
    agg = jax.ops.segment_sum(msg, dst_all, num_segments=NTOT)
    return agg.reshape(NTOT, D)


def kernel(x, n_node_edge_index, Wl, Wr, att, b_gat, W1, b1, W2, b2,
           g1, be1, g2, be2):
    xf = x.reshape(NTOT, D)
    xl, xr = _proj(xf, Wl, Wr)
    agg = _edge_phase(xl, xr, att, n_node_edge_index[0], n_node_edge_index[1])
    out = _ffn(xf, agg, b_gat, W1, b1, W2, b2, g1, be1, g2, be2)
    return out.reshape(B, T, N, D)


# speedup vs baseline: 59.2614x; 58.5953x over previous
import functools

import jax
import jax.numpy as jnp
from jax import lax
from jax.experimental import pallas as pl
from jax.experimental.pallas import tpu as pltpu
from jax.experimental.pallas import tpu_sc as plsc

B, T, N, D = 4, 12, 1000, 64
H, FH = 8, 8
FF = 256
E = 16000
NEG_SLOPE = 0.2
BT = B * T
NTOT = BT * N

_ROWS = NTOT
_BLK = 3000

_HP = 4
_TASKS = BT * _HP
_NW = 32
_TPW = _TASKS // _NW
_NPAD = 1024


def _mm_kernel(x_ref, wl_ref, wr_ref, xl_ref, xr_ref):
    x = x_ref[...]
    dn = (((1,), (0,)), ((), ()))
    xl_ref[...] = lax.dot_general(x, wl_ref[...], dn,
                                  preferred_element_type=jnp.float32,
                                  precision=lax.Precision.HIGHEST)
    xr_ref[...] = lax.dot_general(x, wr_ref[...], dn,
                                  preferred_element_type=jnp.float32,
                                  precision=lax.Precision.HIGHEST)


def _proj(xf, Wl, Wr):
    grid = _ROWS // _BLK
    return pl.pallas_call(
        _mm_kernel,
        grid=(grid,),
        in_specs=[
            pl.BlockSpec((_BLK, D), lambda i: (i, 0)),
            pl.BlockSpec((D, D), lambda i: (0, 0)),
            pl.BlockSpec((D, D), lambda i: (0, 0)),
        ],
        out_specs=[
            pl.BlockSpec((_BLK, D), lambda i: (i, 0)),
            pl.BlockSpec((_BLK, D), lambda i: (i, 0)),
        ],
        out_shape=[
            jax.ShapeDtypeStruct((_ROWS, D), jnp.float32),
            jax.ShapeDtypeStruct((_ROWS, D), jnp.float32),
        ],
    )(xf, Wl, Wr)


def _sc_edge_body(xlt_hbm, xrt_hbm, src_hbm, dst_hbm, att_hbm, agg_hbm,
                  xl_v, xr_v, src_v, dst_v, pb0, pb1, den0, den1, agg_v,
                  att_v):
    wid = lax.axis_index("s") * 2 + lax.axis_index("c")

    pltpu.sync_copy(src_hbm, src_v)
    pltpu.sync_copy(dst_hbm, dst_v)
    pltpu.sync_copy(att_hbm, att_v)

    zeros = jnp.zeros((16,), jnp.float32)

    for t in range(_TPW):
        task = wid * _TPW + t
        hp = lax.rem(task, _HP)

        pltpu.sync_copy(xlt_hbm.at[task], xl_v)
        pltpu.sync_copy(xrt_hbm.at[task], xr_v)

        def _zero_tables(j, _):
            sl16 = pl.ds(j * 16, 16)
            den0[sl16] = zeros
            den1[sl16] = zeros
            agg_v[pl.ds(j * 256, 16)] = zeros
            for k in range(1, 16):
                agg_v[pl.ds(j * 256 + k * 16, 16)] = zeros
            return _
        lax.fori_loop(0, _NPAD // 16, _zero_tables, None)

        att_cols = [att_v[pl.ds(hp * 256 + c * 16, 16)] for c in range(16)]

        def _pass1(g, _):
            sl = pl.ds(g * 16, 16)
            sv = src_v[sl] * 16
            dv = dst_v[sl]
            dv16 = dv * 16
            acc0 = jnp.zeros((16,), jnp.float32)
            acc1 = jnp.zeros((16,), jnp.float32)
            for c in range(16):
                gl = plsc.load_gather(xl_v, [sv + c])
                gr = plsc.load_gather(xr_v, [dv16 + c])
                e = gl + gr
                e = jnp.maximum(e, NEG_SLOPE * e)
                w = att_cols[c] * e
                if c < 8:
                    acc0 = acc0 + w
                else:
                    acc1 = acc1 + w
            p0 = jnp.exp(acc0)
            p1 = jnp.exp(acc1)
            pb0[sl] = p0
            pb1[sl] = p1
            plsc.addupdate_scatter(den0, [dv], p0)
            plsc.addupdate_scatter(den1, [dv], p1)
            return _
        lax.fori_loop(0, E // 16, _pass1, None)

        def _pass2(g, _):
            sl = pl.ds(g * 16, 16)
            sv = src_v[sl] * 16
            dv = dst_v[sl]
            dv16 = dv * 16
            d0 = plsc.load_gather(den0, [dv])
            d1 = plsc.load_gather(den1, [dv])
            a0 = pb0[sl] / (d0 + 1e-16)
            a1 = pb1[sl] / (d1 + 1e-16)
            for c in range(16):
                gl = plsc.load_gather(xl_v, [sv + c])
                a = a0 if c < 8 else a1
                plsc.addupdate_scatter(agg_v, [dv16 + c], gl * a)
            return _
        lax.fori_loop(0, E // 16, _pass2, None)

        pltpu.sync_copy(agg_v.at[pl.ds(0, N * 16)], agg_hbm.at[task])


def _sc_edge(xlt, xrt, src, dst, att_splat):
    mesh = plsc.VectorSubcoreMesh(core_axis_name="c", subcore_axis_name="s")
    f = pl.kernel(
        _sc_edge_body,
        out_type=jax.ShapeDtypeStruct((_TASKS, N * 16), jnp.float32),
        mesh=mesh,
        compiler_params=pltpu.CompilerParams(needs_layout_passes=False),
        scratch_types=[
            pltpu.VMEM((N * 16,), jnp.float32),
            pltpu.VMEM((N * 16,), jnp.float32),
            pltpu.VMEM((E,), jnp.int32),
            pltpu.VMEM((E,), jnp.int32),
            pltpu.VMEM((E,), jnp.float32),
            pltpu.VMEM((E,), jnp.float32),
            pltpu.VMEM((_NPAD,), jnp.float32),
            pltpu.VMEM((_NPAD,), jnp.float32),
            pltpu.VMEM((_NPAD * 16,), jnp.float32),
            pltpu.VMEM((_HP * 256,), jnp.float32),
        ],
    )
    return f(xlt, xrt, src, dst, att_splat)


def _ln(v, g, b):
    mu = jnp.mean(v, axis=-1, keepdims=True)
    var = jnp.mean((v - mu) ** 2, axis=-1, keepdims=True)
    return (v - mu) * lax.rsqrt(var + 1e-5) * g + b


def _ffn_kernel(x_ref, agg_ref, w1_ref, b1_ref, w2_ref, b2_ref,
                g1_ref, be1_ref, g2_ref, be2_ref, bg_ref, o_ref):
    x = x_ref[...]
    dn = (((1,), (0,)), ((), ()))
    out = _ln(x + agg_ref[...] + bg_ref[...], g1_ref[...], be1_ref[...])
    h = lax.dot_general(out, w1_ref[...], dn,
                        preferred_element_type=jnp.float32,
                        precision=lax.Precision.HIGHEST)
    h = jnp.maximum(h + b1_ref[...], 0.0)
    ff = lax.dot_general(h, w2_ref[...], dn,
                         preferred_element_type=jnp.float32,
                         precision=lax.Precision.HIGHEST)
    ff = ff + b2_ref[...]
    o_ref[...] = _ln(out + ff, g2_ref[...], be2_ref[...])


def _ffn(xf, agg, b_gat, W1, b1, W2, b2, g1, be1, g2, be2):
    grid = _ROWS // _BLK
    return pl.pallas_call(
        _ffn_kernel,
        grid=(grid,),
        in_specs=[
            pl.BlockSpec((_BLK, D), lambda i: (i, 0)),
            pl.BlockSpec((_BLK, D), lambda i: (i, 0)),
            pl.BlockSpec((D, FF), lambda i: (0, 0)),
            pl.BlockSpec((1, FF), lambda i: (0, 0)),
            pl.BlockSpec((FF, D), lambda i: (0, 0)),
            pl.BlockSpec((1, D), lambda i: (0, 0)),
            pl.BlockSpec((1, D), lambda i: (0, 0)),
            pl.BlockSpec((1, D), lambda i: (0, 0)),
            pl.BlockSpec((1, D), lambda i: (0, 0)),
            pl.BlockSpec((1, D), lambda i: (0, 0)),
            pl.BlockSpec((1, D), lambda i: (0, 0)),
        ],
        out_specs=pl.BlockSpec((_BLK, D), lambda i: (i, 0)),
        out_shape=jax.ShapeDtypeStruct((_ROWS, D), jnp.float32),
    )(xf, agg, W1, b1.reshape(1, FF), W2, b2.reshape(1, D),
      g1.reshape(1, D), be1.reshape(1, D), g2.reshape(1, D),
      be2.reshape(1, D), b_gat.reshape(1, D))


def kernel(x, n_node_edge_index, Wl, Wr, att, b_gat, W1, b1, W2, b2,
           g1, be1, g2, be2):
    xf = x.reshape(NTOT, D)
    xl, xr = _proj(xf, Wl, Wr)

    xlt = xl.reshape(BT, N, _HP, 16).transpose(0, 2, 1, 3).reshape(_TASKS, N * 16)
    xrt = xr.reshape(BT, N, _HP, 16).transpose(0, 2, 1, 3).reshape(_TASKS, N * 16)
    att_splat = jnp.broadcast_to(
        att.reshape(_HP, 16)[:, :, None], (_HP, 16, 16)).reshape(_HP * 256)
    src = n_node_edge_index[0]
    dst = n_node_edge_index[1]

    aggt = _sc_edge(xlt, xrt, src, dst, att_splat)
    agg = (aggt.reshape(BT, _HP, N, 16).transpose(0, 2, 1, 3)
           .reshape(NTOT, D))

    out = _ffn(xf, agg, b_gat, W1, b1, W2, b2, g1, be1, g2, be2)
    return out.reshape(B, T, N, D)


# speedup vs baseline: 186.1400x; 3.1410x over previous
import functools

import jax
import jax.numpy as jnp
from jax import lax
from jax.experimental import pallas as pl
from jax.experimental.pallas import tpu as pltpu
from jax.experimental.pallas import tpu_sc as plsc

B, T, N, D = 4, 12, 1000, 64
H, FH = 8, 8
FF = 256
E = 16000
NEG_SLOPE = 0.2
BT = B * T
NTOT = BT * N

_ROWS = NTOT
_BLK = 3000

_HP = 4
_TASKS = BT * _HP
_NW = 32
_TPW = _TASKS // _NW
_NPAD = 1024


def _mm_kernel(x_ref, wl_ref, wr_ref, xl_ref, xr_ref):
    x = x_ref[...]
    dn = (((1,), (0,)), ((), ()))
    xl_ref[...] = lax.dot_general(x, wl_ref[...], dn,
                                  preferred_element_type=jnp.float32,
                                  precision=lax.Precision.HIGHEST)
    xr_ref[...] = lax.dot_general(x, wr_ref[...], dn,
                                  preferred_element_type=jnp.float32,
                                  precision=lax.Precision.HIGHEST)


def _proj(xf, Wl, Wr):
    grid = _ROWS // _BLK
    return pl.pallas_call(
        _mm_kernel,
        grid=(grid,),
        in_specs=[
            pl.BlockSpec((_BLK, D), lambda i: (i, 0)),
            pl.BlockSpec((D, D), lambda i: (0, 0)),
            pl.BlockSpec((D, D), lambda i: (0, 0)),
        ],
        out_specs=[
            pl.BlockSpec((_BLK, D), lambda i: (i, 0)),
            pl.BlockSpec((_BLK, D), lambda i: (i, 0)),
        ],
        out_shape=[
            jax.ShapeDtypeStruct((_ROWS, D), jnp.float32),
            jax.ShapeDtypeStruct((_ROWS, D), jnp.float32),
        ],
    )(xf, Wl, Wr)


def _sc_edge_body(xlt_hbm, xrt_hbm, src_hbm, dst_hbm, att_hbm, agg_hbm,
                  xl_v, xr_v, src_v, dst_v, den0, den1, agg_v, att_v):
    wid = lax.axis_index("s") * 2 + lax.axis_index("c")

    pltpu.sync_copy(src_hbm, src_v)
    pltpu.sync_copy(dst_hbm, dst_v)
    pltpu.sync_copy(att_hbm, att_v)

    zeros = jnp.zeros((16,), jnp.float32)

    for t in range(_TPW):
        task = wid * _TPW + t
        hp = lax.rem(task, _HP)

        pltpu.sync_copy(xlt_hbm.at[task], xl_v)
        pltpu.sync_copy(xrt_hbm.at[task], xr_v)

        @plsc.parallel_loop(0, _NPAD, 16)
        def _zero_den(j):
            sl16 = pl.ds(j, 16)
            den0[sl16] = zeros
            den1[sl16] = zeros

        @plsc.parallel_loop(0, _NPAD * 16, 16, unroll=4)
        def _zero_agg(j):
            agg_v[pl.ds(j, 16)] = zeros

        att_cols = [att_v[pl.ds(hp * 256 + c * 16, 16)] for c in range(16)]
        xl_sl = [xl_v.at[pl.ds(c * _NPAD, _NPAD)] for c in range(16)]
        xr_sl = [xr_v.at[pl.ds(c * _NPAD, _NPAD)] for c in range(16)]
        agg_sl = [agg_v.at[pl.ds(c * _NPAD, _NPAD)] for c in range(16)]

        @plsc.parallel_loop(0, E, 16)
        def _edges(i):
            sl = pl.ds(i, 16)
            sv = src_v[sl]
            dv = dst_v[sl]
            acc0 = jnp.zeros((16,), jnp.float32)
            acc1 = jnp.zeros((16,), jnp.float32)
            gls = []
            for c in range(16):
                gl = plsc.load_gather(xl_sl[c], [sv])
                gr = plsc.load_gather(xr_sl[c], [dv])
                e = gl + gr
                e = jnp.maximum(e, NEG_SLOPE * e)
                w = att_cols[c] * e
                if c < 8:
                    acc0 = acc0 + w
                else:
                    acc1 = acc1 + w
                gls.append(gl)
            p0 = jnp.exp(acc0)
            p1 = jnp.exp(acc1)
            plsc.addupdate_scatter(den0, [dv], p0)
            plsc.addupdate_scatter(den1, [dv], p1)
            for c in range(16):
                p = p0 if c < 8 else p1
                plsc.addupdate_scatter(agg_sl[c], [dv], gls[c] * p)

        @plsc.parallel_loop(0, _NPAD, 16)
        def _norm(j):
            dsl = pl.ds(j, 16)
            r0 = 1.0 / (den0[dsl] + 1e-16)
            r1 = 1.0 / (den1[dsl] + 1e-16)
            for c in range(16):
                r = r0 if c < 8 else r1
                off = pl.ds(c * _NPAD + j, 16)
                agg_v[off] = agg_v[off] * r

        pltpu.sync_copy(agg_v, agg_hbm.at[task])


def _sc_edge(xlt, xrt, src, dst, att_splat):
    mesh = plsc.VectorSubcoreMesh(core_axis_name="c", subcore_axis_name="s")
    f = pl.kernel(
        _sc_edge_body,
        out_type=jax.ShapeDtypeStruct((_TASKS, _NPAD * 16), jnp.float32),
        mesh=mesh,
        compiler_params=pltpu.CompilerParams(needs_layout_passes=False),
        scratch_types=[
            pltpu.VMEM((_NPAD * 16,), jnp.float32),
            pltpu.VMEM((_NPAD * 16,), jnp.float32),
            pltpu.VMEM((E,), jnp.int32),
            pltpu.VMEM((E,), jnp.int32),
            pltpu.VMEM((_NPAD,), jnp.float32),
            pltpu.VMEM((_NPAD,), jnp.float32),
            pltpu.VMEM((_NPAD * 16,), jnp.float32),
            pltpu.VMEM((_HP * 256,), jnp.float32),
        ],
    )
    return f(xlt, xrt, src, dst, att_splat)


def _ln(v, g, b):
    mu = jnp.mean(v, axis=-1, keepdims=True)
    var = jnp.mean((v - mu) ** 2, axis=-1, keepdims=True)
    return (v - mu) * lax.rsqrt(var + 1e-5) * g + b


def _ffn_kernel(x_ref, agg_ref, w1_ref, b1_ref, w2_ref, b2_ref,
                g1_ref, be1_ref, g2_ref, be2_ref, bg_ref, o_ref):
    x = x_ref[...]
    dn = (((1,), (0,)), ((), ()))
    out = _ln(x + agg_ref[...] + bg_ref[...], g1_ref[...], be1_ref[...])
    h = lax.dot_general(out, w1_ref[...], dn,
                        preferred_element_type=jnp.float32,
                        precision=lax.Precision.HIGHEST)
    h = jnp.maximum(h + b1_ref[...], 0.0)
    ff = lax.dot_general(h, w2_ref[...], dn,
                         preferred_element_type=jnp.float32,
                         precision=lax.Precision.HIGHEST)
    ff = ff + b2_ref[...]
    o_ref[...] = _ln(out + ff, g2_ref[...], be2_ref[...])


def _ffn(xf, agg, b_gat, W1, b1, W2, b2, g1, be1, g2, be2):
    grid = _ROWS // _BLK
    return pl.pallas_call(
        _ffn_kernel,
        grid=(grid,),
        in_specs=[
            pl.BlockSpec((_BLK, D), lambda i: (i, 0)),
            pl.BlockSpec((_BLK, D), lambda i: (i, 0)),
            pl.BlockSpec((D, FF), lambda i: (0, 0)),
            pl.BlockSpec((1, FF), lambda i: (0, 0)),
            pl.BlockSpec((FF, D), lambda i: (0, 0)),
            pl.BlockSpec((1, D), lambda i: (0, 0)),
            pl.BlockSpec((1, D), lambda i: (0, 0)),
            pl.BlockSpec((1, D), lambda i: (0, 0)),
            pl.BlockSpec((1, D), lambda i: (0, 0)),
            pl.BlockSpec((1, D), lambda i: (0, 0)),
            pl.BlockSpec((1, D), lambda i: (0, 0)),
        ],
        out_specs=pl.BlockSpec((_BLK, D), lambda i: (i, 0)),
        out_shape=jax.ShapeDtypeStruct((_ROWS, D), jnp.float32),
    )(xf, agg, W1, b1.reshape(1, FF), W2, b2.reshape(1, D),
      g1.reshape(1, D), be1.reshape(1, D), g2.reshape(1, D),
      be2.reshape(1, D), b_gat.reshape(1, D))


def kernel(x, n_node_edge_index, Wl, Wr, att, b_gat, W1, b1, W2, b2,
           g1, be1, g2, be2):
    xf = x.reshape(NTOT, D)
    xl, xr = _proj(xf, Wl, Wr)

    def _to_tasks_cm(v):
        v = v.reshape(BT, N, _HP, 16).transpose(0, 2, 3, 1)
        v = jnp.pad(v, ((0, 0), (0, 0), (0, 0), (0, _NPAD - N)))
        return v.reshape(_TASKS, 16 * _NPAD)

    xlt = _to_tasks_cm(xl)
    xrt = _to_tasks_cm(xr)
    att_splat = jnp.broadcast_to(
        att.reshape(_HP, 16)[:, :, None], (_HP, 16, 16)).reshape(_HP * 256)
    src = n_node_edge_index[0]
    dst = n_node_edge_index[1]

    aggt = _sc_edge(xlt, xrt, src, dst, att_splat)
    agg = (aggt.reshape(BT, _HP, 16, _NPAD)[:, :, :, :N]
           .transpose(0, 3, 1, 2).reshape(NTOT, D))

    out = _ffn(xf, agg, b_gat, W1, b1, W2, b2, g1, be1, g2, be2)
    return out.reshape(B, T, N, D)


# speedup vs baseline: 204.5713x; 1.0990x over previous
import jax
import jax.numpy as jnp
from jax import lax
from jax.experimental import pallas as pl
from jax.experimental.pallas import tpu as pltpu
from jax.experimental.pallas import tpu_sc as plsc

B, T, N, D = 4, 12, 1000, 64
H, FH = 8, 8
FF = 256
E = 16000
NEG_SLOPE = 0.2
BT = B * T
NTOT = BT * N

_HP = 4
_TASKS = BT * _HP
_TPW = _TASKS // 32
_NPAD = 1024

_HIGH = lax.Precision.HIGHEST


def _mm_kernel(x_ref, wl_ref, wr_ref, xl_ref, xr_ref):
    x = x_ref[0]
    pad = jnp.zeros((D, _NPAD - N), jnp.float32)
    dn = (((0,), (1,)), ((), ()))
    ylt = lax.dot_general(wl_ref[...], x, dn,
                          preferred_element_type=jnp.float32,
                          precision=_HIGH)
    yrt = lax.dot_general(wr_ref[...], x, dn,
                          preferred_element_type=jnp.float32,
                          precision=_HIGH)
    xl_ref[0] = jnp.concatenate([ylt, pad], axis=1)
    xr_ref[0] = jnp.concatenate([yrt, pad], axis=1)


def _proj(x3, Wl, Wr):
    return pl.pallas_call(
        _mm_kernel,
        grid=(BT,),
        in_specs=[
            pl.BlockSpec((1, N, D), lambda i: (i, 0, 0)),
            pl.BlockSpec((D, D), lambda i: (0, 0)),
            pl.BlockSpec((D, D), lambda i: (0, 0)),
        ],
        out_specs=[
            pl.BlockSpec((1, D, _NPAD), lambda i: (i, 0, 0)),
            pl.BlockSpec((1, D, _NPAD), lambda i: (i, 0, 0)),
        ],
        out_shape=[
            jax.ShapeDtypeStruct((BT, D, _NPAD), jnp.float32),
            jax.ShapeDtypeStruct((BT, D, _NPAD), jnp.float32),
        ],
    )(x3, Wl, Wr)


def _sc_edge_body(xlt_hbm, xrt_hbm, src_hbm, dst_hbm, att_hbm, agg_hbm,
                  xl_v, xr_v, src_v, dst_v, den0, den1, agg_v, att_v):
    wid = lax.axis_index("s") * 2 + lax.axis_index("c")

    pltpu.sync_copy(src_hbm, src_v)
    pltpu.sync_copy(dst_hbm, dst_v)
    pltpu.sync_copy(att_hbm, att_v)

    zeros = jnp.zeros((16,), jnp.float32)

    for t in range(_TPW):
        task = wid * _TPW + t
        bt = task // _HP
        hp16 = lax.rem(task, _HP) * 16

        pltpu.sync_copy(xlt_hbm.at[bt, pl.ds(hp16 * _NPAD, 16 * _NPAD)], xl_v)
        pltpu.sync_copy(xrt_hbm.at[bt, pl.ds(hp16 * _NPAD, 16 * _NPAD)], xr_v)

        @plsc.parallel_loop(0, _NPAD, 16)
        def _zero_den(j):
            sl16 = pl.ds(j, 16)
            den0[sl16] = zeros
            den1[sl16] = zeros

        @plsc.parallel_loop(0, _NPAD * 16, 16, unroll=4)
        def _zero_agg(j):
            agg_v[pl.ds(j, 16)] = zeros

        att_cols = [plsc.load_gather(att_v, [jnp.full((16,), 1, jnp.int32)
                                             * (hp16 + c)])
                    for c in range(16)]
        xl_sl = [xl_v.at[pl.ds(c * _NPAD, _NPAD)] for c in range(16)]
        xr_sl = [xr_v.at[pl.ds(c * _NPAD, _NPAD)] for c in range(16)]
        agg_sl = [agg_v.at[pl.ds(c * _NPAD, _NPAD)] for c in range(16)]

        @plsc.parallel_loop(0, E, 16)
        def _edges(i):
            sl = pl.ds(i, 16)
            sv = src_v[sl]
            dv = dst_v[sl]
            acc0 = jnp.zeros((16,), jnp.float32)
            acc1 = jnp.zeros((16,), jnp.float32)
            gls = []
            for c in range(16):
                gl = plsc.load_gather(xl_sl[c], [sv])
                gr = plsc.load_gather(xr_sl[c], [dv])
                e = gl + gr
                e = jnp.maximum(e, NEG_SLOPE * e)
                w = att_cols[c] * e
                if c < 8:
                    acc0 = acc0 + w
                else:
                    acc1 = acc1 + w
                gls.append(gl)
            p0 = jnp.exp(acc0)
            p1 = jnp.exp(acc1)
            plsc.addupdate_scatter(den0, [dv], p0)
            plsc.addupdate_scatter(den1, [dv], p1)
            for c in range(16):
                p = p0 if c < 8 else p1
                plsc.addupdate_scatter(agg_sl[c], [dv], gls[c] * p)

        @plsc.parallel_loop(0, _NPAD, 16)
        def _norm(j):
            dsl = pl.ds(j, 16)
            r0 = 1.0 / (den0[dsl] + 1e-16)
            r1 = 1.0 / (den1[dsl] + 1e-16)
            for c in range(16):
                r = r0 if c < 8 else r1
                off = pl.ds(c * _NPAD + j, 16)
                agg_v[off] = agg_v[off] * r

        pltpu.sync_copy(agg_v, agg_hbm.at[bt, pl.ds(hp16 * _NPAD, 16 * _NPAD)])


def _sc_edge(xlt, xrt, src, dst, att64):
    mesh = plsc.VectorSubcoreMesh(core_axis_name="c", subcore_axis_name="s")
    f = pl.kernel(
        _sc_edge_body,
        out_type=jax.ShapeDtypeStruct((BT, D * _NPAD), jnp.float32),
        mesh=mesh,
        compiler_params=pltpu.CompilerParams(needs_layout_passes=False),
        scratch_types=[
            pltpu.VMEM((_NPAD * 16,), jnp.float32),
            pltpu.VMEM((_NPAD * 16,), jnp.float32),
            pltpu.VMEM((E,), jnp.int32),
            pltpu.VMEM((E,), jnp.int32),
            pltpu.VMEM((_NPAD,), jnp.float32),
            pltpu.VMEM((_NPAD,), jnp.float32),
            pltpu.VMEM((_NPAD * 16,), jnp.float32),
            pltpu.VMEM((D,), jnp.float32),
        ],
    )
    return f(xlt.reshape(BT, D * _NPAD), xrt.reshape(BT, D * _NPAD),
             src, dst, att64)


def _ln(v, g, b):
    mu = jnp.mean(v, axis=-1, keepdims=True)
    var = jnp.mean((v - mu) ** 2, axis=-1, keepdims=True)
    return (v - mu) * lax.rsqrt(var + 1e-5) * g + b


def _ffn_kernel(x_ref, agg_ref, w1_ref, b1_ref, w2_ref, b2_ref,
                g1_ref, be1_ref, g2_ref, be2_ref, bg_ref, o_ref):
    x = x_ref[0]
    agg_cm = agg_ref[0]
    eye = (jax.lax.broadcasted_iota(jnp.int32, (D, D), 0) ==
           jax.lax.broadcasted_iota(jnp.int32, (D, D), 1)
           ).astype(jnp.float32)
    dn = (((0,), (0,)), ((), ()))
    agg = lax.dot_general(agg_cm[:, :N], eye, dn,
                          preferred_element_type=jnp.float32,
                          precision=_HIGH)
    dn2 = (((1,), (0,)), ((), ()))
    out = _ln(x + agg + bg_ref[...], g1_ref[...], be1_ref[...])
    h = lax.dot_general(out, w1_ref[...], dn2,
                        preferred_element_type=jnp.float32, precision=_HIGH)
    h = jnp.maximum(h + b1_ref[...], 0.0)
    ff = lax.dot_general(h, w2_ref[...], dn2,
                         preferred_element_type=jnp.float32, precision=_HIGH)
    ff = ff + b2_ref[...]
    o_ref[0] = _ln(out + ff, g2_ref[...], be2_ref[...])


def _ffn(x3, aggt, b_gat, W1, b1, W2, b2, g1, be1, g2, be2):
    vec = lambda n: pl.BlockSpec((1, n), lambda i: (0, 0))
    return pl.pallas_call(
        _ffn_kernel,
        grid=(BT,),
        in_specs=[
            pl.BlockSpec((1, N, D), lambda i: (i, 0, 0)),
            pl.BlockSpec((1, D, _NPAD), lambda i: (i, 0, 0)),
            pl.BlockSpec((D, FF), lambda i: (0, 0)),
            vec(FF),
            pl.BlockSpec((FF, D), lambda i: (0, 0)),
            vec(D), vec(D), vec(D), vec(D), vec(D), vec(D),
        ],
        out_specs=pl.BlockSpec((1, N, D), lambda i: (i, 0, 0)),
        out_shape=jax.ShapeDtypeStruct((BT, N, D), jnp.float32),
    )(x3, aggt, W1, b1.reshape(1, FF), W2, b2.reshape(1, D),
      g1.reshape(1, D), be1.reshape(1, D), g2.reshape(1, D),
      be2.reshape(1, D), b_gat.reshape(1, D))


def kernel(x, n_node_edge_index, Wl, Wr, att, b_gat, W1, b1, W2, b2,
           g1, be1, g2, be2):
    x3 = x.reshape(BT, N, D)
    xlt, xrt = _proj(x3, Wl, Wr)
    aggt = _sc_edge(xlt, xrt, n_node_edge_index[0], n_node_edge_index[1],
                    att.reshape(D))
    out = _ffn(x3, aggt.reshape(BT, D, _NPAD), b_gat, W1, b1, W2, b2,
               g1, be1, g2, be2)
    return out.reshape(B, T, N, D)


# speedup vs baseline: 225.3448x; 1.1015x over previous
import jax
import jax.numpy as jnp
from jax import lax
from jax.experimental import pallas as pl
from jax.experimental.pallas import tpu as pltpu
from jax.experimental.pallas import tpu_sc as plsc

B, T, N, D = 4, 12, 1000, 64
H, FH = 8, 8
FF = 256
E = 16000
NEG_SLOPE = 0.2
BT = B * T
NTOT = BT * N

_HP = 4
_TASKS = BT * _HP
_TPW = _TASKS // 32
_NPAD = 1024

_HIGH = lax.Precision.HIGHEST


def _mm_kernel(x_ref, wl_ref, wr_ref, xl_ref, xr_ref):
    x = x_ref[0]
    pad = jnp.zeros((D, _NPAD - N), jnp.float32)
    dn = (((0,), (1,)), ((), ()))
    ylt = lax.dot_general(wl_ref[...], x, dn,
                          preferred_element_type=jnp.float32,
                          precision=_HIGH)
    yrt = lax.dot_general(wr_ref[...], x, dn,
                          preferred_element_type=jnp.float32,
                          precision=_HIGH)
    xl_ref[0] = jnp.concatenate([ylt, pad], axis=1)
    xr_ref[0] = jnp.concatenate([yrt, pad], axis=1)


def _proj(x3, Wl, Wr):
    return pl.pallas_call(
        _mm_kernel,
        grid=(BT,),
        in_specs=[
            pl.BlockSpec((1, N, D), lambda i: (i, 0, 0)),
            pl.BlockSpec((D, D), lambda i: (0, 0)),
            pl.BlockSpec((D, D), lambda i: (0, 0)),
        ],
        out_specs=[
            pl.BlockSpec((1, D, _NPAD), lambda i: (i, 0, 0)),
            pl.BlockSpec((1, D, _NPAD), lambda i: (i, 0, 0)),
        ],
        out_shape=[
            jax.ShapeDtypeStruct((BT, D, _NPAD), jnp.float32),
            jax.ShapeDtypeStruct((BT, D, _NPAD), jnp.float32),
        ],
    )(x3, Wl, Wr)


def _sc_edge_body(xlt_hbm, xrt_hbm, ei_hbm, att_hbm, agg_hbm,
                  xl_v, xr_v, src_v, dst_v, den0, den1, agg_v, att_v):
    wid = lax.axis_index("s") * 2 + lax.axis_index("c")

    pltpu.sync_copy(ei_hbm.at[0], src_v)
    pltpu.sync_copy(ei_hbm.at[1], dst_v)
    pltpu.sync_copy(att_hbm, att_v)

    zeros = jnp.zeros((16,), jnp.float32)

    for t in range(_TPW):
        task = wid * _TPW + t
        bt = task // _HP
        hp16 = lax.rem(task, _HP) * 16

        pltpu.sync_copy(xlt_hbm.at[bt, pl.ds(hp16 * _NPAD, 16 * _NPAD)], xl_v)
        pltpu.sync_copy(xrt_hbm.at[bt, pl.ds(hp16 * _NPAD, 16 * _NPAD)], xr_v)

        @plsc.parallel_loop(0, _NPAD, 16)
        def _zero_den(j):
            sl16 = pl.ds(j, 16)
            den0[sl16] = zeros
            den1[sl16] = zeros

        @plsc.parallel_loop(0, _NPAD * 16, 16, unroll=4)
        def _zero_agg(j):
            agg_v[pl.ds(j, 16)] = zeros

        att_cols = [plsc.load_gather(att_v, [jnp.full((16,), 1, jnp.int32)
                                             * (hp16 + c)])
                    for c in range(16)]
        xl_sl = [xl_v.at[pl.ds(c * _NPAD, _NPAD)] for c in range(16)]
        xr_sl = [xr_v.at[pl.ds(c * _NPAD, _NPAD)] for c in range(16)]
        agg_sl = [agg_v.at[pl.ds(c * _NPAD, _NPAD)] for c in range(16)]

        dens = (den0, den1)

        @plsc.parallel_loop(0, E, 16, unroll=2)
        def _edges(i):
            sl = pl.ds(i, 16)
            sv = src_v[sl]
            dv = dst_v[sl]
            for h in range(2):
                acc = jnp.zeros((16,), jnp.float32)
                gls = []
                for k in range(8):
                    c = h * 8 + k
                    gl = plsc.load_gather(xl_sl[c], [sv])
                    gr = plsc.load_gather(xr_sl[c], [dv])
                    e = gl + gr
                    e = jnp.maximum(e, NEG_SLOPE * e)
                    acc = acc + att_cols[c] * e
                    gls.append(gl)
                p = jnp.exp(acc)
                plsc.addupdate_scatter(dens[h], [dv], p)
                for k in range(8):
                    plsc.addupdate_scatter(agg_sl[h * 8 + k], [dv],
                                           gls[k] * p)

        @plsc.parallel_loop(0, _NPAD, 16)
        def _norm(j):
            dsl = pl.ds(j, 16)
            r0 = 1.0 / (den0[dsl] + 1e-16)
            r1 = 1.0 / (den1[dsl] + 1e-16)
            for c in range(16):
                r = r0 if c < 8 else r1
                off = pl.ds(c * _NPAD + j, 16)
                agg_v[off] = agg_v[off] * r

        pltpu.sync_copy(agg_v, agg_hbm.at[bt, pl.ds(hp16 * _NPAD, 16 * _NPAD)])


def _sc_edge(xlt, xrt, ei, att64):
    mesh = plsc.VectorSubcoreMesh(core_axis_name="c", subcore_axis_name="s")
    f = pl.kernel(
        _sc_edge_body,
        out_type=jax.ShapeDtypeStruct((BT, D * _NPAD), jnp.float32),
        mesh=mesh,
        compiler_params=pltpu.CompilerParams(needs_layout_passes=False),
        scratch_types=[
            pltpu.VMEM((_NPAD * 16,), jnp.float32),
            pltpu.VMEM((_NPAD * 16,), jnp.float32),
            pltpu.VMEM((E,), jnp.int32),
            pltpu.VMEM((E,), jnp.int32),
            pltpu.VMEM((_NPAD,), jnp.float32),
            pltpu.VMEM((_NPAD,), jnp.float32),
            pltpu.VMEM((_NPAD * 16,), jnp.float32),
            pltpu.VMEM((D,), jnp.float32),
        ],
    )
    return f(xlt.reshape(BT, D * _NPAD), xrt.reshape(BT, D * _NPAD),
             ei, att64)


def _ln(v, g, b):
    mu = jnp.mean(v, axis=-1, keepdims=True)
    var = jnp.mean((v - mu) ** 2, axis=-1, keepdims=True)
    return (v - mu) * lax.rsqrt(var + 1e-5) * g + b


def _ffn_kernel(x_ref, agg_ref, w1_ref, b1_ref, w2_ref, b2_ref,
                g1_ref, be1_ref, g2_ref, be2_ref, bg_ref, o_ref):
    x = x_ref[0]
    agg_cm = agg_ref[0]
    eye = (jax.lax.broadcasted_iota(jnp.int32, (D, D), 0) ==
           jax.lax.broadcasted_iota(jnp.int32, (D, D), 1)
           ).astype(jnp.float32)
    dn = (((0,), (0,)), ((), ()))
    agg = lax.dot_general(agg_cm[:, :N], eye, dn,
                          preferred_element_type=jnp.float32,
                          precision=_HIGH)
    dn2 = (((1,), (0,)), ((), ()))
    out = _ln(x + agg + bg_ref[...], g1_ref[...], be1_ref[...])
    h = lax.dot_general(out, w1_ref[...], dn2,
                        preferred_element_type=jnp.float32, precision=_HIGH)
    h = jnp.maximum(h + b1_ref[...], 0.0)
    ff = lax.dot_general(h, w2_ref[...], dn2,
                         preferred_element_type=jnp.float32, precision=_HIGH)
    ff = ff + b2_ref[...]
    o_ref[0] = _ln(out + ff, g2_ref[...], be2_ref[...])


def _ffn(x3, aggt, b_gat, W1, b1, W2, b2, g1, be1, g2, be2):
    vec = lambda n: pl.BlockSpec((1, n), lambda i: (0, 0))
    return pl.pallas_call(
        _ffn_kernel,
        grid=(BT,),
        in_specs=[
            pl.BlockSpec((1, N, D), lambda i: (i, 0, 0)),
            pl.BlockSpec((1, D, _NPAD), lambda i: (i, 0, 0)),
            pl.BlockSpec((D, FF), lambda i: (0, 0)),
            vec(FF),
            pl.BlockSpec((FF, D), lambda i: (0, 0)),
            vec(D), vec(D), vec(D), vec(D), vec(D), vec(D),
        ],
        out_specs=pl.BlockSpec((1, N, D), lambda i: (i, 0, 0)),
        out_shape=jax.ShapeDtypeStruct((BT, N, D), jnp.float32),
    )(x3, aggt, W1, b1.reshape(1, FF), W2, b2.reshape(1, D),
      g1.reshape(1, D), be1.reshape(1, D), g2.reshape(1, D),
      be2.reshape(1, D), b_gat.reshape(1, D))


def kernel(x, n_node_edge_index, Wl, Wr, att, b_gat, W1, b1, W2, b2,
           g1, be1, g2, be2):
    x3 = x.reshape(BT, N, D)
    xlt, xrt = _proj(x3, Wl, Wr)
    aggt = _sc_edge(xlt, xrt, n_node_edge_index, att.reshape(D))
    out = _ffn(x3, aggt.reshape(BT, D, _NPAD), b_gat, W1, b1, W2, b2,
               g1, be1, g2, be2)
    return out.reshape(B, T, N, D)
